# baseline (speedup 1.0000x reference)
import functools

import jax
import jax.numpy as jnp
from jax import lax
from jax.experimental import pallas as pl
from jax.experimental.pallas import tpu as pltpu
from jax.experimental.pallas import tpu_sc as plsc

D = 768
E = 16
F = 1152
T = 4096
TM = 128
NT = T // TM
MAX_WORK = NT + E - 1
PAD_BLOCKS = MAX_WORK + 1
PAD_ROWS = PAD_BLOCKS * TM

NW = 32
TPW = T // NW


def _route_body(x_ref, wr_ref, e_ref, r_ref, cnt_ref, carry_ref):
    i = pl.program_id(0)

    @pl.when(i == 0)
    def _():
        carry_ref[...] = jnp.zeros_like(carry_ref)

    xb = x_ref[...]
    logits = lax.dot_general(
        xb.astype(jnp.bfloat16), wr_ref[...].astype(jnp.bfloat16),
        (((1,), (1,)), ((), ())),
        preferred_element_type=jnp.float32)
    m = jnp.max(logits, axis=1, keepdims=True)
    lanes = lax.broadcasted_iota(jnp.int32, (TM, E), 1)
    cand = jnp.where(logits >= m, lanes, E)
    e_col = jnp.min(cand, axis=1, keepdims=True)
    onehot = (lanes == e_col).astype(jnp.float32)

    row_i = lax.broadcasted_iota(jnp.int32, (TM, TM), 0)
    col_i = lax.broadcasted_iota(jnp.int32, (TM, TM), 1)
    tri = (row_i > col_i).astype(jnp.float32)
    excl = lax.dot_general(
        tri, onehot, (((1,), (0,)), ((), ())),
        preferred_element_type=jnp.float32)

    carry = carry_ref[0:1, :]
    rank = (jnp.sum(excl * onehot, axis=1, keepdims=True)
            + jnp.sum(carry * onehot, axis=1, keepdims=True))

    e_ref[0] = e_col.astype(jnp.int32)
    r_ref[0] = rank.astype(jnp.int32)

    new_carry = carry + jnp.sum(onehot, axis=0, keepdims=True)
    carry_ref[0:1, :] = new_carry
    cnt_ref[0] = new_carry.astype(jnp.int32)


def _route(x2, Wr):
    return pl.pallas_call(
        _route_body,
        grid=(NT,),
        in_specs=[
            pl.BlockSpec((TM, D), lambda i: (i, 0)),
            pl.BlockSpec((E, D), lambda i: (0, 0)),
        ],
        out_specs=[
            pl.BlockSpec((1, TM, 1), lambda i: (i, 0, 0)),
            pl.BlockSpec((1, TM, 1), lambda i: (i, 0, 0)),
            pl.BlockSpec((1, 1, E), lambda i: (0, 0, 0)),
        ],
        out_shape=[
            jax.ShapeDtypeStruct((NT, TM, 1), jnp.int32),
            jax.ShapeDtypeStruct((NT, TM, 1), jnp.int32),
            jax.ShapeDtypeStruct((1, 1, E), jnp.int32),
        ],
        scratch_shapes=[pltpu.VMEM((8, E), jnp.float32)],
    )(x2, Wr)


def _dest_body(e_ref, r_ref, po_ref, d_ref):
    lanes = lax.broadcasted_iota(jnp.int32, (TM, E), 1)
    onehot = (lanes == e_ref[0]).astype(jnp.int32)
    sel = jnp.sum(onehot * po_ref[0], axis=1, keepdims=True)
    d_ref[0] = sel + r_ref[0]


def _dest(e3, r3, po3):
    return pl.pallas_call(
        _dest_body,
        grid=(NT,),
        in_specs=[
            pl.BlockSpec((1, TM, 1), lambda i: (i, 0, 0)),
            pl.BlockSpec((1, TM, 1), lambda i: (i, 0, 0)),
            pl.BlockSpec((1, 1, E), lambda i: (0, 0, 0)),
        ],
        out_specs=pl.BlockSpec((1, TM, 1), lambda i: (i, 0, 0)),
        out_shape=jax.ShapeDtypeStruct((NT, TM, 1), jnp.int32),
    )(e3, r3, po3)


def _sc_mesh():
    return plsc.VectorSubcoreMesh(core_axis_name="c", subcore_axis_name="s")


def _dispatch(x2, dest):
    @functools.partial(
        pl.kernel,
        mesh=_sc_mesh(),
        out_type=jax.ShapeDtypeStruct((PAD_ROWS, D), jnp.float32),
        scratch_types=[
            pltpu.VMEM((TPW,), jnp.int32),
            pltpu.VMEM((TPW, D), jnp.float32),
            pltpu.SemaphoreType.DMA,
        ],
    )
    def disp(x_hbm, dest_hbm, xpad_hbm, idx_v, rows_v, sem):
        wid = lax.axis_index("s") * 2 + lax.axis_index("c")
        base = wid * TPW
        pltpu.sync_copy(dest_hbm.at[pl.ds(base, TPW)], idx_v)
        pltpu.sync_copy(x_hbm.at[pl.ds(base, TPW)], rows_v)
        pltpu.async_copy(rows_v, xpad_hbm.at[idx_v], sem).wait()

    return disp(x2, dest)


def _ffn_body(es_ref, bs_ref, x_ref, wg_ref, wu_ref, wd_ref, o_ref):
    i = pl.program_id(0)

    @pl.when(bs_ref[i] < MAX_WORK)
    def _():
        xb = x_ref[...].astype(jnp.bfloat16)
        wg = wg_ref[0].astype(jnp.bfloat16)
        wu = wu_ref[0].astype(jnp.bfloat16)
        g = lax.dot_general(xb, wg, (((1,), (1,)), ((), ())),
                            preferred_element_type=jnp.float32)
        u = lax.dot_general(xb, wu, (((1,), (1,)), ((), ())),
                            preferred_element_type=jnp.float32)
        h = (g * jax.nn.sigmoid(g) * u).astype(jnp.bfloat16)
        wd = wd_ref[0].astype(jnp.bfloat16)
        o_ref[...] = lax.dot_general(h, wd, (((1,), (1,)), ((), ())),
                                     preferred_element_type=jnp.float32)


def _ffn(es, bs, xpad, Wg, Wu, Wd):
    grid_spec = pltpu.PrefetchScalarGridSpec(
        num_scalar_prefetch=2,
        grid=(MAX_WORK,),
        in_specs=[
            pl.BlockSpec((TM, D), lambda i, es, bs: (bs[i], 0)),
            pl.BlockSpec((1, F, D), lambda i, es, bs: (es[i], 0, 0)),
            pl.BlockSpec((1, F, D), lambda i, es, bs: (es[i], 0, 0)),
            pl.BlockSpec((1, D, F), lambda i, es, bs: (es[i], 0, 0)),
        ],
        out_specs=pl.BlockSpec((TM, D), lambda i, es, bs: (bs[i], 0)),
    )
    return pl.pallas_call(
        _ffn_body,
        grid_spec=grid_spec,
        out_shape=jax.ShapeDtypeStruct((PAD_ROWS, D), jnp.float32),
    )(es, bs, xpad, Wg, Wu, Wd)


def _combine(outpad, dest):
    @functools.partial(
        pl.kernel,
        mesh=_sc_mesh(),
        out_type=jax.ShapeDtypeStruct((T, D), jnp.float32),
        scratch_types=[
            pltpu.VMEM((TPW,), jnp.int32),
            pltpu.VMEM((TPW, D), jnp.float32),
            pltpu.SemaphoreType.DMA,
        ],
    )
    def comb(opad_hbm, dest_hbm, out_hbm, idx_v, rows_v, sem):
        wid = lax.axis_index("s") * 2 + lax.axis_index("c")
        base = wid * TPW
        pltpu.sync_copy(dest_hbm.at[pl.ds(base, TPW)], idx_v)
        pltpu.async_copy(opad_hbm.at[idx_v], rows_v, sem).wait()
        pltpu.sync_copy(rows_v, out_hbm.at[pl.ds(base, TPW)])

    return comb(outpad, dest)


def kernel(x, Wr, Wg, Wu, Wd):
    B, L, _ = x.shape
    x2 = x.reshape(T, D)
    e3, r3, cnt3 = _route(x2, Wr)
    counts = cnt3.reshape(E)

    ntiles = (counts + TM - 1) // TM
    end_blk = jnp.cumsum(ntiles)
    total = end_blk[E - 1]
    steps = jnp.arange(MAX_WORK, dtype=jnp.int32)
    raw = jnp.sum((steps[:, None] >= end_blk[None, :]).astype(jnp.int32), axis=1)
    es = jnp.minimum(raw, E - 1).astype(jnp.int32)
    bs = jnp.where(steps < total, steps, MAX_WORK).astype(jnp.int32)
    pad_off = ((end_blk - ntiles) * TM).astype(jnp.int32)

    dest3 = _dest(e3, r3, pad_off.reshape(1, 1, E))
    dest = dest3.reshape(T)
    xpad = _dispatch(x2, dest)
    outpad = _ffn(es, bs, xpad, Wg, Wu, Wd)
    out2 = _combine(outpad, dest)
    return out2.reshape(B, L, D)


# speedup vs baseline: 1.2209x; 1.2209x over previous
import functools

import jax
import jax.numpy as jnp
from jax import lax
from jax.experimental import pallas as pl
from jax.experimental.pallas import tpu as pltpu
from jax.experimental.pallas import tpu_sc as plsc

D = 768
E = 16
F = 1152
T = 4096
TM = 128
NT = T // TM
TMF = 256
MAX_WORK = T // TMF + E - 1
PAD_BLOCKS = MAX_WORK + 1
PAD_ROWS = PAD_BLOCKS * TMF

NW = 32
TPW = T // NW


def _route_body(x_ref, wr_ref, e_ref, r_ref, cnt_ref, carry_ref):
    i = pl.program_id(0)

    @pl.when(i == 0)
    def _():
        carry_ref[...] = jnp.zeros_like(carry_ref)

    xb = x_ref[...]
    logits = lax.dot_general(
        xb.astype(jnp.bfloat16), wr_ref[...].astype(jnp.bfloat16),
        (((1,), (1,)), ((), ())),
        preferred_element_type=jnp.float32)
    m = jnp.max(logits, axis=1, keepdims=True)
    lanes = lax.broadcasted_iota(jnp.int32, (TM, E), 1)
    cand = jnp.where(logits >= m, lanes, E)
    e_col = jnp.min(cand, axis=1, keepdims=True)
    onehot = (lanes == e_col).astype(jnp.float32)

    row_i = lax.broadcasted_iota(jnp.int32, (TM, TM), 0)
    col_i = lax.broadcasted_iota(jnp.int32, (TM, TM), 1)
    tri = (row_i > col_i).astype(jnp.float32)
    excl = lax.dot_general(
        tri, onehot, (((1,), (0,)), ((), ())),
        preferred_element_type=jnp.float32)

    carry = carry_ref[0:1, :]
    rank = (jnp.sum(excl * onehot, axis=1, keepdims=True)
            + jnp.sum(carry * onehot, axis=1, keepdims=True))

    e_ref[0] = e_col.astype(jnp.int32)
    r_ref[0] = rank.astype(jnp.int32)

    new_carry = carry + jnp.sum(onehot, axis=0, keepdims=True)
    carry_ref[0:1, :] = new_carry
    cnt_ref[0] = new_carry.astype(jnp.int32)


def _route(x2, Wr):
    return pl.pallas_call(
        _route_body,
        grid=(NT,),
        in_specs=[
            pl.BlockSpec((TM, D), lambda i: (i, 0)),
            pl.BlockSpec((E, D), lambda i: (0, 0)),
        ],
        out_specs=[
            pl.BlockSpec((1, TM, 1), lambda i: (i, 0, 0)),
            pl.BlockSpec((1, TM, 1), lambda i: (i, 0, 0)),
            pl.BlockSpec((1, 1, E), lambda i: (0, 0, 0)),
        ],
        out_shape=[
            jax.ShapeDtypeStruct((NT, TM, 1), jnp.int32),
            jax.ShapeDtypeStruct((NT, TM, 1), jnp.int32),
            jax.ShapeDtypeStruct((1, 1, E), jnp.int32),
        ],
        scratch_shapes=[pltpu.VMEM((8, E), jnp.float32)],
    )(x2, Wr)


def _dest_body(e_ref, r_ref, po_ref, d_ref):
    lanes = lax.broadcasted_iota(jnp.int32, (TM, E), 1)
    onehot = (lanes == e_ref[0]).astype(jnp.int32)
    sel = jnp.sum(onehot * po_ref[0], axis=1, keepdims=True)
    d_ref[0] = sel + r_ref[0]


def _dest(e3, r3, po3):
    return pl.pallas_call(
        _dest_body,
        grid=(NT,),
        in_specs=[
            pl.BlockSpec((1, TM, 1), lambda i: (i, 0, 0)),
            pl.BlockSpec((1, TM, 1), lambda i: (i, 0, 0)),
            pl.BlockSpec((1, 1, E), lambda i: (0, 0, 0)),
        ],
        out_specs=pl.BlockSpec((1, TM, 1), lambda i: (i, 0, 0)),
        out_shape=jax.ShapeDtypeStruct((NT, TM, 1), jnp.int32),
    )(e3, r3, po3)


def _sc_mesh():
    return plsc.VectorSubcoreMesh(core_axis_name="c", subcore_axis_name="s")


def _dispatch(x2, dest):
    @functools.partial(
        pl.kernel,
        mesh=_sc_mesh(),
        out_type=jax.ShapeDtypeStruct((PAD_ROWS, D), jnp.float32),
        scratch_types=[
            pltpu.VMEM((TPW,), jnp.int32),
            pltpu.VMEM((TPW, D), jnp.float32),
            pltpu.SemaphoreType.DMA,
        ],
    )
    def disp(x_hbm, dest_hbm, xpad_hbm, idx_v, rows_v, sem):
        wid = lax.axis_index("s") * 2 + lax.axis_index("c")
        base = wid * TPW
        pltpu.sync_copy(dest_hbm.at[pl.ds(base, TPW)], idx_v)
        pltpu.sync_copy(x_hbm.at[pl.ds(base, TPW)], rows_v)
        pltpu.async_copy(rows_v, xpad_hbm.at[idx_v], sem).wait()

    return disp(x2, dest)


def _ffn_body(es_ref, bs_ref, x_ref, wg_ref, wu_ref, wd_ref, o_ref):
    i = pl.program_id(0)

    @pl.when(bs_ref[i] < MAX_WORK)
    def _():
        xb = x_ref[...].astype(jnp.bfloat16)
        wg = wg_ref[0].astype(jnp.bfloat16)
        wu = wu_ref[0].astype(jnp.bfloat16)
        g = lax.dot_general(xb, wg, (((1,), (1,)), ((), ())),
                            preferred_element_type=jnp.float32)
        u = lax.dot_general(xb, wu, (((1,), (1,)), ((), ())),
                            preferred_element_type=jnp.float32)
        h = (g * jax.nn.sigmoid(g) * u).astype(jnp.bfloat16)
        wd = wd_ref[0].astype(jnp.bfloat16)
        o_ref[...] = lax.dot_general(h, wd, (((1,), (1,)), ((), ())),
                                     preferred_element_type=jnp.float32)


def _ffn(es, bs, xpad, Wg, Wu, Wd):
    grid_spec = pltpu.PrefetchScalarGridSpec(
        num_scalar_prefetch=2,
        grid=(MAX_WORK,),
        in_specs=[
            pl.BlockSpec((TMF, D), lambda i, es, bs: (bs[i], 0)),
            pl.BlockSpec((1, F, D), lambda i, es, bs: (es[i], 0, 0)),
            pl.BlockSpec((1, F, D), lambda i, es, bs: (es[i], 0, 0)),
            pl.BlockSpec((1, D, F), lambda i, es, bs: (es[i], 0, 0)),
        ],
        out_specs=pl.BlockSpec((TMF, D), lambda i, es, bs: (bs[i], 0)),
    )
    return pl.pallas_call(
        _ffn_body,
        grid_spec=grid_spec,
        out_shape=jax.ShapeDtypeStruct((PAD_ROWS, D), jnp.float32),
    )(es, bs, xpad, Wg, Wu, Wd)


def _combine(outpad, dest):
    @functools.partial(
        pl.kernel,
        mesh=_sc_mesh(),
        out_type=jax.ShapeDtypeStruct((T, D), jnp.float32),
        scratch_types=[
            pltpu.VMEM((TPW,), jnp.int32),
            pltpu.VMEM((TPW, D), jnp.float32),
            pltpu.SemaphoreType.DMA,
        ],
    )
    def comb(opad_hbm, dest_hbm, out_hbm, idx_v, rows_v, sem):
        wid = lax.axis_index("s") * 2 + lax.axis_index("c")
        base = wid * TPW
        pltpu.sync_copy(dest_hbm.at[pl.ds(base, TPW)], idx_v)
        pltpu.async_copy(opad_hbm.at[idx_v], rows_v, sem).wait()
        pltpu.sync_copy(rows_v, out_hbm.at[pl.ds(base, TPW)])

    return comb(outpad, dest)


def kernel(x, Wr, Wg, Wu, Wd):
    B, L, _ = x.shape
    x2 = x.reshape(T, D)
    e3, r3, cnt3 = _route(x2, Wr)
    counts = cnt3.reshape(E)

    ntiles = (counts + TMF - 1) // TMF
    end_blk = jnp.cumsum(ntiles)
    total = end_blk[E - 1]
    steps = jnp.arange(MAX_WORK, dtype=jnp.int32)
    raw = jnp.sum((steps[:, None] >= end_blk[None, :]).astype(jnp.int32), axis=1)
    es = jnp.minimum(raw, E - 1).astype(jnp.int32)
    bs = jnp.where(steps < total, steps, MAX_WORK).astype(jnp.int32)
    pad_off = ((end_blk - ntiles) * TMF).astype(jnp.int32)

    dest3 = _dest(e3, r3, pad_off.reshape(1, 1, E))
    dest = dest3.reshape(T)
    xpad = _dispatch(x2, dest)
    outpad = _ffn(es, bs, xpad, Wg, Wu, Wd)
    out2 = _combine(outpad, dest)
    return out2.reshape(B, L, D)


# speedup vs baseline: 1.2924x; 1.0586x over previous
import functools

import jax
import jax.numpy as jnp
from jax import lax
from jax.experimental import pallas as pl
from jax.experimental.pallas import tpu as pltpu
from jax.experimental.pallas import tpu_sc as plsc

D = 768
E = 16
F = 1152
T = 4096
TM = 128
NT = T // TM
TMF = 256
MAX_WORK = T // TMF + E - 1
PAD_BLOCKS = MAX_WORK + 1
PAD_ROWS = PAD_BLOCKS * TMF

NW = 32
TPW = T // NW


def _route_body(x_ref, wr_ref, d_ref, es_ref, bs_ref,
                carry_ref, e_s, r_s, po_s):
    p = pl.program_id(0)
    i = pl.program_id(1)

    @pl.when(jnp.logical_and(p == 0, i == 0))
    def _():
        carry_ref[...] = jnp.zeros_like(carry_ref)

    lanes = lax.broadcasted_iota(jnp.int32, (TM, E), 1)

    @pl.when(p == 0)
    def _():
        xb = x_ref[...]
        logits = lax.dot_general(
            xb.astype(jnp.bfloat16), wr_ref[...].astype(jnp.bfloat16),
            (((1,), (1,)), ((), ())),
            preferred_element_type=jnp.float32)
        m = jnp.max(logits, axis=1, keepdims=True)
        cand = jnp.where(logits >= m, lanes, E)
        e_col = jnp.min(cand, axis=1, keepdims=True)
        onehot = (lanes == e_col).astype(jnp.float32)

        row_i = lax.broadcasted_iota(jnp.int32, (TM, TM), 0)
        col_i = lax.broadcasted_iota(jnp.int32, (TM, TM), 1)
        tri = (row_i > col_i).astype(jnp.float32)
        excl = lax.dot_general(
            tri, onehot, (((1,), (0,)), ((), ())),
            preferred_element_type=jnp.float32)

        carry = carry_ref[0:1, :]
        rank = (jnp.sum(excl * onehot, axis=1, keepdims=True)
                + jnp.sum(carry * onehot, axis=1, keepdims=True))

        e_s[pl.ds(i * TM, TM), :] = e_col.astype(jnp.int32)
        r_s[pl.ds(i * TM, TM), :] = rank.astype(jnp.int32)
        carry_ref[0:1, :] = carry + jnp.sum(onehot, axis=0, keepdims=True)

    @pl.when(jnp.logical_and(p == 1, i == 0))
    def _():
        counts = carry_ref[0:1, :]
        nt = jnp.floor((counts + (TMF - 1)) * (1.0 / TMF))
        k_i = lax.broadcasted_iota(jnp.int32, (E, E), 0)
        j_i = lax.broadcasted_iota(jnp.int32, (E, E), 1)
        le_r = (k_i <= j_i).astype(jnp.float32)
        end_row = lax.dot_general(nt, le_r, (((1,), (0,)), ((), ())),
                                  preferred_element_type=jnp.float32)
        po_s[0:1, :] = (end_row - nt) * TMF
        end_col = lax.dot_general(le_r, nt, (((0,), (1,)), ((), ())),
                                  preferred_element_type=jnp.float32)
        total = jnp.max(end_col)
        s_iota = lax.broadcasted_iota(jnp.int32, (E, MAX_WORK), 1)
        cmp = (s_iota >= end_col.astype(jnp.int32)).astype(jnp.int32)
        raw = jnp.sum(cmp, axis=0, keepdims=True)
        es_ref[0] = jnp.minimum(raw, E - 1)
        s_row = lax.broadcasted_iota(jnp.int32, (1, MAX_WORK), 1)
        bs_ref[0] = jnp.where(s_row < total.astype(jnp.int32), s_row, MAX_WORK)

    @pl.when(p == 1)
    def _():
        e_col = e_s[pl.ds(i * TM, TM), :]
        r_col = r_s[pl.ds(i * TM, TM), :]
        onehot = (lanes == e_col).astype(jnp.int32)
        po = po_s[0:1, :].astype(jnp.int32)
        d_ref[0] = jnp.sum(onehot * po, axis=1, keepdims=True) + r_col


def _route(x2, Wr):
    return pl.pallas_call(
        _route_body,
        grid=(2, NT),
        in_specs=[
            pl.BlockSpec((TM, D), lambda p, i: (i * (1 - p) + (NT - 1) * p, 0)),
            pl.BlockSpec((E, D), lambda p, i: (0, 0)),
        ],
        out_specs=[
            pl.BlockSpec((1, TM, 1), lambda p, i: (p * i, 0, 0)),
            pl.BlockSpec((1, 1, MAX_WORK), lambda p, i: (0, 0, 0)),
            pl.BlockSpec((1, 1, MAX_WORK), lambda p, i: (0, 0, 0)),
        ],
        out_shape=[
            jax.ShapeDtypeStruct((NT, TM, 1), jnp.int32),
            jax.ShapeDtypeStruct((1, 1, MAX_WORK), jnp.int32),
            jax.ShapeDtypeStruct((1, 1, MAX_WORK), jnp.int32),
        ],
        scratch_shapes=[
            pltpu.VMEM((8, E), jnp.float32),
            pltpu.VMEM((T, 1), jnp.int32),
            pltpu.VMEM((T, 1), jnp.int32),
            pltpu.VMEM((8, E), jnp.float32),
        ],
    )(x2, Wr)


def _sc_mesh():
    return plsc.VectorSubcoreMesh(core_axis_name="c", subcore_axis_name="s")


def _dispatch(x2, dest):
    @functools.partial(
        pl.kernel,
        mesh=_sc_mesh(),
        out_type=jax.ShapeDtypeStruct((PAD_ROWS, D), jnp.float32),
        scratch_types=[
            pltpu.VMEM((TPW,), jnp.int32),
            pltpu.VMEM((TPW, D), jnp.float32),
            pltpu.SemaphoreType.DMA,
        ],
    )
    def disp(x_hbm, dest_hbm, xpad_hbm, idx_v, rows_v, sem):
        wid = lax.axis_index("s") * 2 + lax.axis_index("c")
        base = wid * TPW
        pltpu.sync_copy(dest_hbm.at[pl.ds(base, TPW)], idx_v)
        pltpu.sync_copy(x_hbm.at[pl.ds(base, TPW)], rows_v)
        pltpu.async_copy(rows_v, xpad_hbm.at[idx_v], sem).wait()

    return disp(x2, dest)


def _ffn_body(es_ref, bs_ref, x_ref, wg_ref, wu_ref, wd_ref, o_ref):
    i = pl.program_id(0)

    @pl.when(bs_ref[i] < MAX_WORK)
    def _():
        xb = x_ref[...].astype(jnp.bfloat16)
        wg = wg_ref[0].astype(jnp.bfloat16)
        wu = wu_ref[0].astype(jnp.bfloat16)
        g = lax.dot_general(xb, wg, (((1,), (1,)), ((), ())),
                            preferred_element_type=jnp.float32)
        u = lax.dot_general(xb, wu, (((1,), (1,)), ((), ())),
                            preferred_element_type=jnp.float32)
        h = (g * jax.nn.sigmoid(g) * u).astype(jnp.bfloat16)
        wd = wd_ref[0].astype(jnp.bfloat16)
        o_ref[...] = lax.dot_general(h, wd, (((1,), (1,)), ((), ())),
                                     preferred_element_type=jnp.float32)


def _ffn(es, bs, xpad, Wg, Wu, Wd):
    grid_spec = pltpu.PrefetchScalarGridSpec(
        num_scalar_prefetch=2,
        grid=(MAX_WORK,),
        in_specs=[
            pl.BlockSpec((TMF, D), lambda i, es, bs: (bs[i], 0)),
            pl.BlockSpec((1, F, D), lambda i, es, bs: (es[i], 0, 0)),
            pl.BlockSpec((1, F, D), lambda i, es, bs: (es[i], 0, 0)),
            pl.BlockSpec((1, D, F), lambda i, es, bs: (es[i], 0, 0)),
        ],
        out_specs=pl.BlockSpec((TMF, D), lambda i, es, bs: (bs[i], 0)),
    )
    return pl.pallas_call(
        _ffn_body,
        grid_spec=grid_spec,
        out_shape=jax.ShapeDtypeStruct((PAD_ROWS, D), jnp.float32),
    )(es, bs, xpad, Wg, Wu, Wd)


def _combine(outpad, dest):
    @functools.partial(
        pl.kernel,
        mesh=_sc_mesh(),
        out_type=jax.ShapeDtypeStruct((T, D), jnp.float32),
        scratch_types=[
            pltpu.VMEM((TPW,), jnp.int32),
            pltpu.VMEM((TPW, D), jnp.float32),
            pltpu.SemaphoreType.DMA,
        ],
    )
    def comb(opad_hbm, dest_hbm, out_hbm, idx_v, rows_v, sem):
        wid = lax.axis_index("s") * 2 + lax.axis_index("c")
        base = wid * TPW
        pltpu.sync_copy(dest_hbm.at[pl.ds(base, TPW)], idx_v)
        pltpu.async_copy(opad_hbm.at[idx_v], rows_v, sem).wait()
        pltpu.sync_copy(rows_v, out_hbm.at[pl.ds(base, TPW)])

    return comb(outpad, dest)


def kernel(x, Wr, Wg, Wu, Wd):
    B, L, _ = x.shape
    x2 = x.reshape(T, D)
    dest3, es3, bs3 = _route(x2, Wr)
    dest = dest3.reshape(T)
    es = es3.reshape(MAX_WORK)
    bs = bs3.reshape(MAX_WORK)
    xpad = _dispatch(x2, dest)
    outpad = _ffn(es, bs, xpad, Wg, Wu, Wd)
    out2 = _combine(outpad, dest)
    return out2.reshape(B, L, D)


# speedup vs baseline: 1.4722x; 1.1392x over previous
import functools

import jax
import jax.numpy as jnp
from jax import lax
from jax.experimental import pallas as pl
from jax.experimental.pallas import tpu as pltpu
from jax.experimental.pallas import tpu_sc as plsc

D = 768
E = 16
F = 1152
T = 4096
TM = 512
NT = T // TM
TMF = 256
MAX_WORK = T // TMF + E - 1
PAD_BLOCKS = MAX_WORK + 1
PAD_ROWS = PAD_BLOCKS * TMF

NW = 32
TPW = T // NW


def _route_body(x_ref, wr_ref, d_ref, es_ref, bs_ref,
                carry_ref, e_s, r_s, po_s):
    p = pl.program_id(0)
    i = pl.program_id(1)

    @pl.when(jnp.logical_and(p == 0, i == 0))
    def _():
        carry_ref[...] = jnp.zeros_like(carry_ref)

    lanes = lax.broadcasted_iota(jnp.int32, (TM, E), 1)

    @pl.when(p == 0)
    def _():
        xb = x_ref[...]
        logits = lax.dot_general(
            xb.astype(jnp.bfloat16), wr_ref[...].astype(jnp.bfloat16),
            (((1,), (1,)), ((), ())),
            preferred_element_type=jnp.float32)
        m = jnp.max(logits, axis=1, keepdims=True)
        cand = jnp.where(logits >= m, lanes, E)
        e_col = jnp.min(cand, axis=1, keepdims=True)
        onehot = (lanes == e_col).astype(jnp.float32)

        row_i = lax.broadcasted_iota(jnp.int32, (TM, TM), 0)
        col_i = lax.broadcasted_iota(jnp.int32, (TM, TM), 1)
        tri = (row_i > col_i).astype(jnp.float32)
        excl = lax.dot_general(
            tri, onehot, (((1,), (0,)), ((), ())),
            preferred_element_type=jnp.float32,
            precision=lax.Precision.HIGHEST)

        carry = carry_ref[0:1, :]
        rank = (jnp.sum(excl * onehot, axis=1, keepdims=True)
                + jnp.sum(carry * onehot, axis=1, keepdims=True))

        e_s[pl.ds(i * TM, TM), :] = e_col.astype(jnp.int32)
        r_s[pl.ds(i * TM, TM), :] = rank.astype(jnp.int32)
        carry_ref[0:1, :] = carry + jnp.sum(onehot, axis=0, keepdims=True)

    @pl.when(jnp.logical_and(p == 1, i == 0))
    def _():
        counts = carry_ref[0:1, :]
        nt = jnp.floor((counts + (TMF - 1)) * (1.0 / TMF))
        k_i = lax.broadcasted_iota(jnp.int32, (E, E), 0)
        j_i = lax.broadcasted_iota(jnp.int32, (E, E), 1)
        le_r = (k_i <= j_i).astype(jnp.float32)
        end_row = lax.dot_general(nt, le_r, (((1,), (0,)), ((), ())),
                                  preferred_element_type=jnp.float32)
        po_s[0:1, :] = (end_row - nt) * TMF
        end_col = lax.dot_general(le_r, nt, (((0,), (1,)), ((), ())),
                                  preferred_element_type=jnp.float32)
        total = jnp.max(end_col)
        s_iota = lax.broadcasted_iota(jnp.int32, (E, MAX_WORK), 1)
        cmp = (s_iota >= end_col.astype(jnp.int32)).astype(jnp.int32)
        raw = jnp.sum(cmp, axis=0, keepdims=True)
        es_ref[0] = jnp.minimum(raw, E - 1)
        s_row = lax.broadcasted_iota(jnp.int32, (1, MAX_WORK), 1)
        bs_ref[0] = jnp.where(s_row < total.astype(jnp.int32), s_row, MAX_WORK)

    @pl.when(p == 1)
    def _():
        e_col = e_s[pl.ds(i * TM, TM), :]
        r_col = r_s[pl.ds(i * TM, TM), :]
        onehot = (lanes == e_col).astype(jnp.int32)
        po = po_s[0:1, :].astype(jnp.int32)
        d_ref[0] = jnp.sum(onehot * po, axis=1, keepdims=True) + r_col


def _route(x2, Wr):
    return pl.pallas_call(
        _route_body,
        grid=(2, NT),
        in_specs=[
            pl.BlockSpec((TM, D), lambda p, i: (i * (1 - p) + (NT - 1) * p, 0)),
            pl.BlockSpec((E, D), lambda p, i: (0, 0)),
        ],
        out_specs=[
            pl.BlockSpec((1, TM, 1), lambda p, i: (p * i, 0, 0)),
            pl.BlockSpec((1, 1, MAX_WORK), lambda p, i: (0, 0, 0)),
            pl.BlockSpec((1, 1, MAX_WORK), lambda p, i: (0, 0, 0)),
        ],
        out_shape=[
            jax.ShapeDtypeStruct((NT, TM, 1), jnp.int32),
            jax.ShapeDtypeStruct((1, 1, MAX_WORK), jnp.int32),
            jax.ShapeDtypeStruct((1, 1, MAX_WORK), jnp.int32),
        ],
        scratch_shapes=[
            pltpu.VMEM((8, E), jnp.float32),
            pltpu.VMEM((T, 1), jnp.int32),
            pltpu.VMEM((T, 1), jnp.int32),
            pltpu.VMEM((8, E), jnp.float32),
        ],
    )(x2, Wr)


def _sc_mesh():
    return plsc.VectorSubcoreMesh(core_axis_name="c", subcore_axis_name="s")


def _dispatch(x2, dest):
    @functools.partial(
        pl.kernel,
        mesh=_sc_mesh(),
        out_type=jax.ShapeDtypeStruct((PAD_ROWS, D), jnp.float32),
        scratch_types=[
            pltpu.VMEM((TPW,), jnp.int32),
            pltpu.VMEM((TPW, D), jnp.float32),
            pltpu.SemaphoreType.DMA,
        ],
    )
    def disp(x_hbm, dest_hbm, xpad_hbm, idx_v, rows_v, sem):
        wid = lax.axis_index("s") * 2 + lax.axis_index("c")
        base = wid * TPW
        pltpu.sync_copy(dest_hbm.at[pl.ds(base, TPW)], idx_v)
        pltpu.sync_copy(x_hbm.at[pl.ds(base, TPW)], rows_v)
        pltpu.async_copy(rows_v, xpad_hbm.at[idx_v], sem).wait()

    return disp(x2, dest)


def _ffn_body(es_ref, bs_ref, x_ref, wg_ref, wu_ref, wd_ref, o_ref):
    i = pl.program_id(0)

    @pl.when(bs_ref[i] < MAX_WORK)
    def _():
        xb = x_ref[...].astype(jnp.bfloat16)
        wg = wg_ref[0].astype(jnp.bfloat16)
        wu = wu_ref[0].astype(jnp.bfloat16)
        g = lax.dot_general(xb, wg, (((1,), (1,)), ((), ())),
                            preferred_element_type=jnp.float32)
        u = lax.dot_general(xb, wu, (((1,), (1,)), ((), ())),
                            preferred_element_type=jnp.float32)
        h = (g * jax.nn.sigmoid(g) * u).astype(jnp.bfloat16)
        wd = wd_ref[0].astype(jnp.bfloat16)
        o_ref[...] = lax.dot_general(h, wd, (((1,), (1,)), ((), ())),
                                     preferred_element_type=jnp.float32)


def _ffn(es, bs, xpad, Wg, Wu, Wd):
    grid_spec = pltpu.PrefetchScalarGridSpec(
        num_scalar_prefetch=2,
        grid=(MAX_WORK,),
        in_specs=[
            pl.BlockSpec((TMF, D), lambda i, es, bs: (bs[i], 0)),
            pl.BlockSpec((1, F, D), lambda i, es, bs: (es[i], 0, 0)),
            pl.BlockSpec((1, F, D), lambda i, es, bs: (es[i], 0, 0)),
            pl.BlockSpec((1, D, F), lambda i, es, bs: (es[i], 0, 0)),
        ],
        out_specs=pl.BlockSpec((TMF, D), lambda i, es, bs: (bs[i], 0)),
    )
    return pl.pallas_call(
        _ffn_body,
        grid_spec=grid_spec,
        out_shape=jax.ShapeDtypeStruct((PAD_ROWS, D), jnp.float32),
    )(es, bs, xpad, Wg, Wu, Wd)


def _combine(outpad, dest):
    @functools.partial(
        pl.kernel,
        mesh=_sc_mesh(),
        out_type=jax.ShapeDtypeStruct((T, D), jnp.float32),
        scratch_types=[
            pltpu.VMEM((TPW,), jnp.int32),
            pltpu.VMEM((TPW, D), jnp.float32),
            pltpu.SemaphoreType.DMA,
        ],
    )
    def comb(opad_hbm, dest_hbm, out_hbm, idx_v, rows_v, sem):
        wid = lax.axis_index("s") * 2 + lax.axis_index("c")
        base = wid * TPW
        pltpu.sync_copy(dest_hbm.at[pl.ds(base, TPW)], idx_v)
        pltpu.async_copy(opad_hbm.at[idx_v], rows_v, sem).wait()
        pltpu.sync_copy(rows_v, out_hbm.at[pl.ds(base, TPW)])

    return comb(outpad, dest)


def kernel(x, Wr, Wg, Wu, Wd):
    B, L, _ = x.shape
    x2 = x.reshape(T, D)
    dest3, es3, bs3 = _route(x2, Wr)
    dest = dest3.reshape(T)
    es = es3.reshape(MAX_WORK)
    bs = bs3.reshape(MAX_WORK)
    xpad = _dispatch(x2, dest)
    outpad = _ffn(es, bs, xpad, Wg, Wu, Wd)
    out2 = _combine(outpad, dest)
    return out2.reshape(B, L, D)


# speedup vs baseline: 1.5259x; 1.0365x over previous
import functools

import jax
import jax.numpy as jnp
from jax import lax
from jax.experimental import pallas as pl
from jax.experimental.pallas import tpu as pltpu
from jax.experimental.pallas import tpu_sc as plsc

D = 768
E = 16
F = 1152
T = 4096
TM = 512
NT = T // TM
TMF = 512
MAX_WORK = T // TMF + E - 1
PAD_BLOCKS = MAX_WORK + 1
PAD_ROWS = PAD_BLOCKS * TMF

NW = 32
TPW = T // NW


def _route_body(x_ref, wr_ref, d_ref, es_ref, bs_ref,
                carry_ref, e_s, r_s, po_s):
    p = pl.program_id(0)
    i = pl.program_id(1)

    @pl.when(jnp.logical_and(p == 0, i == 0))
    def _():
        carry_ref[...] = jnp.zeros_like(carry_ref)

    lanes = lax.broadcasted_iota(jnp.int32, (TM, E), 1)

    @pl.when(p == 0)
    def _():
        xb = x_ref[...]
        logits = lax.dot_general(
            xb.astype(jnp.bfloat16), wr_ref[...].astype(jnp.bfloat16),
            (((1,), (1,)), ((), ())),
            preferred_element_type=jnp.float32)
        m = jnp.max(logits, axis=1, keepdims=True)
        cand = jnp.where(logits >= m, lanes, E)
        e_col = jnp.min(cand, axis=1, keepdims=True)
        onehot = (lanes == e_col).astype(jnp.float32)

        row_i = lax.broadcasted_iota(jnp.int32, (TM, TM), 0)
        col_i = lax.broadcasted_iota(jnp.int32, (TM, TM), 1)
        tri = (row_i > col_i).astype(jnp.float32)
        excl = lax.dot_general(
            tri, onehot, (((1,), (0,)), ((), ())),
            preferred_element_type=jnp.float32,
            precision=lax.Precision.HIGHEST)

        carry = carry_ref[0:1, :]
        rank = (jnp.sum(excl * onehot, axis=1, keepdims=True)
                + jnp.sum(carry * onehot, axis=1, keepdims=True))

        e_s[pl.ds(i * TM, TM), :] = e_col.astype(jnp.int32)
        r_s[pl.ds(i * TM, TM), :] = rank.astype(jnp.int32)
        carry_ref[0:1, :] = carry + jnp.sum(onehot, axis=0, keepdims=True)

    @pl.when(jnp.logical_and(p == 1, i == 0))
    def _():
        counts = carry_ref[0:1, :]
        nt = jnp.floor((counts + (TMF - 1)) * (1.0 / TMF))
        k_i = lax.broadcasted_iota(jnp.int32, (E, E), 0)
        j_i = lax.broadcasted_iota(jnp.int32, (E, E), 1)
        le_r = (k_i <= j_i).astype(jnp.float32)
        end_row = lax.dot_general(nt, le_r, (((1,), (0,)), ((), ())),
                                  preferred_element_type=jnp.float32)
        po_s[0:1, :] = (end_row - nt) * TMF
        end_col = lax.dot_general(le_r, nt, (((0,), (1,)), ((), ())),
                                  preferred_element_type=jnp.float32)
        total = jnp.max(end_col)
        s_iota = lax.broadcasted_iota(jnp.int32, (E, MAX_WORK), 1)
        cmp = (s_iota >= end_col.astype(jnp.int32)).astype(jnp.int32)
        raw = jnp.sum(cmp, axis=0, keepdims=True)
        es_ref[0] = jnp.minimum(raw, E - 1)
        s_row = lax.broadcasted_iota(jnp.int32, (1, MAX_WORK), 1)
        bs_ref[0] = jnp.where(s_row < total.astype(jnp.int32), s_row, MAX_WORK)

    @pl.when(p == 1)
    def _():
        e_col = e_s[pl.ds(i * TM, TM), :]
        r_col = r_s[pl.ds(i * TM, TM), :]
        onehot = (lanes == e_col).astype(jnp.int32)
        po = po_s[0:1, :].astype(jnp.int32)
        d_ref[0] = jnp.sum(onehot * po, axis=1, keepdims=True) + r_col


def _route(x2, Wr):
    return pl.pallas_call(
        _route_body,
        grid=(2, NT),
        in_specs=[
            pl.BlockSpec((TM, D), lambda p, i: (i * (1 - p) + (NT - 1) * p, 0)),
            pl.BlockSpec((E, D), lambda p, i: (0, 0)),
        ],
        out_specs=[
            pl.BlockSpec((1, TM, 1), lambda p, i: (p * i, 0, 0)),
            pl.BlockSpec((1, 1, MAX_WORK), lambda p, i: (0, 0, 0)),
            pl.BlockSpec((1, 1, MAX_WORK), lambda p, i: (0, 0, 0)),
        ],
        out_shape=[
            jax.ShapeDtypeStruct((NT, TM, 1), jnp.int32),
            jax.ShapeDtypeStruct((1, 1, MAX_WORK), jnp.int32),
            jax.ShapeDtypeStruct((1, 1, MAX_WORK), jnp.int32),
        ],
        scratch_shapes=[
            pltpu.VMEM((8, E), jnp.float32),
            pltpu.VMEM((T, 1), jnp.int32),
            pltpu.VMEM((T, 1), jnp.int32),
            pltpu.VMEM((8, E), jnp.float32),
        ],
    )(x2, Wr)


def _sc_mesh():
    return plsc.VectorSubcoreMesh(core_axis_name="c", subcore_axis_name="s")


def _dispatch(x2, dest):
    @functools.partial(
        pl.kernel,
        mesh=_sc_mesh(),
        out_type=jax.ShapeDtypeStruct((PAD_ROWS, D), jnp.float32),
        scratch_types=[
            pltpu.VMEM((TPW,), jnp.int32),
            pltpu.VMEM((TPW, D), jnp.float32),
            pltpu.SemaphoreType.DMA,
        ],
    )
    def disp(x_hbm, dest_hbm, xpad_hbm, idx_v, rows_v, sem):
        wid = lax.axis_index("s") * 2 + lax.axis_index("c")
        base = wid * TPW
        pltpu.sync_copy(dest_hbm.at[pl.ds(base, TPW)], idx_v)
        pltpu.sync_copy(x_hbm.at[pl.ds(base, TPW)], rows_v)
        pltpu.async_copy(rows_v, xpad_hbm.at[idx_v], sem).wait()

    return disp(x2, dest)


def _ffn_body(es_ref, bs_ref, x_ref, wg_ref, wu_ref, wd_ref, o_ref):
    i = pl.program_id(0)

    @pl.when(bs_ref[i] < MAX_WORK)
    def _():
        xb = x_ref[...].astype(jnp.bfloat16)
        wg = wg_ref[0].astype(jnp.bfloat16)
        wu = wu_ref[0].astype(jnp.bfloat16)
        g = lax.dot_general(xb, wg, (((1,), (1,)), ((), ())),
                            preferred_element_type=jnp.float32)
        u = lax.dot_general(xb, wu, (((1,), (1,)), ((), ())),
                            preferred_element_type=jnp.float32)
        h = (g * jax.nn.sigmoid(g) * u).astype(jnp.bfloat16)
        wd = wd_ref[0].astype(jnp.bfloat16)
        o_ref[...] = lax.dot_general(h, wd, (((1,), (1,)), ((), ())),
                                     preferred_element_type=jnp.float32)


def _ffn(es, bs, xpad, Wg, Wu, Wd):
    grid_spec = pltpu.PrefetchScalarGridSpec(
        num_scalar_prefetch=2,
        grid=(MAX_WORK,),
        in_specs=[
            pl.BlockSpec((TMF, D), lambda i, es, bs: (bs[i], 0)),
            pl.BlockSpec((1, F, D), lambda i, es, bs: (es[i], 0, 0)),
            pl.BlockSpec((1, F, D), lambda i, es, bs: (es[i], 0, 0)),
            pl.BlockSpec((1, D, F), lambda i, es, bs: (es[i], 0, 0)),
        ],
        out_specs=pl.BlockSpec((TMF, D), lambda i, es, bs: (bs[i], 0)),
    )
    return pl.pallas_call(
        _ffn_body,
        grid_spec=grid_spec,
        out_shape=jax.ShapeDtypeStruct((PAD_ROWS, D), jnp.float32),
    )(es, bs, xpad, Wg, Wu, Wd)


def _combine(outpad, dest):
    @functools.partial(
        pl.kernel,
        mesh=_sc_mesh(),
        out_type=jax.ShapeDtypeStruct((T, D), jnp.float32),
        scratch_types=[
            pltpu.VMEM((TPW,), jnp.int32),
            pltpu.VMEM((TPW, D), jnp.float32),
            pltpu.SemaphoreType.DMA,
        ],
    )
    def comb(opad_hbm, dest_hbm, out_hbm, idx_v, rows_v, sem):
        wid = lax.axis_index("s") * 2 + lax.axis_index("c")
        base = wid * TPW
        pltpu.sync_copy(dest_hbm.at[pl.ds(base, TPW)], idx_v)
        pltpu.async_copy(opad_hbm.at[idx_v], rows_v, sem).wait()
        pltpu.sync_copy(rows_v, out_hbm.at[pl.ds(base, TPW)])

    return comb(outpad, dest)


def kernel(x, Wr, Wg, Wu, Wd):
    B, L, _ = x.shape
    x2 = x.reshape(T, D)
    dest3, es3, bs3 = _route(x2, Wr)
    dest = dest3.reshape(T)
    es = es3.reshape(MAX_WORK)
    bs = bs3.reshape(MAX_WORK)
    xpad = _dispatch(x2, dest)
    outpad = _ffn(es, bs, xpad, Wg, Wu, Wd)
    out2 = _combine(outpad, dest)
    return out2.reshape(B, L, D)
